# baseline (speedup 1.0000x reference)
import functools

import jax
import jax.numpy as jnp
from jax import lax
from jax.experimental import pallas as pl
from jax.experimental.pallas import tpu as pltpu
from jax.experimental.pallas import tpu_sc as plsc

E = 8
D_IN = 2048
Z_DIM = 2048
B = 128

ROWS = E * Z_DIM
ROW_BLK = 1024


COL_BLK = D_IN // 2


def _matvec_body(w_ref, z_ref, b_ref, y_ref):
    j = pl.program_id(1)
    part = jnp.dot(w_ref[...], z_ref[...], preferred_element_type=jnp.float32)

    @pl.when(j == 0)
    def _():
        y_ref[...] = part + b_ref[...]

    @pl.when(j != 0)
    def _():
        y_ref[...] += part


_matvec = pl.pallas_call(
    _matvec_body,
    grid=(ROWS // ROW_BLK, D_IN // COL_BLK),
    in_specs=[
        pl.BlockSpec((ROW_BLK, COL_BLK), lambda i, j: (i, j)),
        pl.BlockSpec((COL_BLK, 1), lambda i, j: (j, 0)),
        pl.BlockSpec((ROW_BLK, 1), lambda i, j: (i, 0)),
    ],
    out_specs=pl.BlockSpec((ROW_BLK, 1), lambda i, j: (i, 0)),
    out_shape=jax.ShapeDtypeStruct((ROWS, 1), jnp.float32),
)


_B_PER_W = 4

_sc_mesh = plsc.VectorSubcoreMesh(core_axis_name="c", subcore_axis_name="s")


@functools.partial(
    pl.kernel,
    out_type=jax.ShapeDtypeStruct((B, Z_DIM), jnp.float32),
    mesh=_sc_mesh,
    scratch_types=[
        pltpu.VMEM((_B_PER_W,), jnp.int32),
        pltpu.VMEM((_B_PER_W, Z_DIM), jnp.float32),
        pltpu.SemaphoreType.DMA,
    ],
)
def _sc_gather(y_hbm, idxpad_hbm, out_hbm, idx_v, rows_v, sem):
    wid = lax.axis_index("s") * 2 + lax.axis_index("c")
    pltpu.sync_copy(idxpad_hbm.at[pl.ds(wid * 8, _B_PER_W)], idx_v)
    pltpu.async_copy(y_hbm.at[idx_v], rows_v, sem).wait()
    pltpu.sync_copy(rows_v, out_hbm.at[pl.ds(wid * _B_PER_W, _B_PER_W)])


def kernel(indices, z, W, b):
    idx = indices.astype(jnp.int32)
    w_flat = W.reshape(ROWS, D_IN)
    b_flat = b.reshape(ROWS, 1)
    z_col = z.reshape(D_IN, 1)
    idx_pad = jnp.pad(idx.reshape(32, _B_PER_W),
                      ((0, 0), (0, 8 - _B_PER_W))).reshape(-1)
    y = _matvec(w_flat, z_col, b_flat).reshape(E, Z_DIM)
    return _sc_gather(y, idx_pad)


# speedup vs baseline: 1.1110x; 1.1110x over previous
import functools

import jax
import jax.numpy as jnp
from jax import lax
from jax.experimental import pallas as pl
from jax.experimental.pallas import tpu as pltpu
from jax.experimental.pallas import tpu_sc as plsc

E = 8
D_IN = 2048
Z_DIM = 2048
B = 128

ROWS = E * Z_DIM
ROW_BLK = 1024


def _matvec_body(w_ref, z_ref, b_ref, y_ref):
    y_ref[...] = (
        jnp.dot(w_ref[...], z_ref[...], preferred_element_type=jnp.float32)
        + b_ref[...]
    )


_matvec = pl.pallas_call(
    _matvec_body,
    grid=(ROWS // ROW_BLK,),
    in_specs=[
        pl.BlockSpec((ROW_BLK, D_IN), lambda i: (i, 0)),
        pl.BlockSpec((D_IN, 1), lambda i: (0, 0)),
        pl.BlockSpec((ROW_BLK, 1), lambda i: (i, 0)),
    ],
    out_specs=pl.BlockSpec((ROW_BLK, 1), lambda i: (i, 0)),
    out_shape=jax.ShapeDtypeStruct((ROWS, 1), jnp.float32),
)


_B_PER_W = 4

_sc_mesh = plsc.VectorSubcoreMesh(core_axis_name="c", subcore_axis_name="s")


@functools.partial(
    pl.kernel,
    out_type=jax.ShapeDtypeStruct((B, Z_DIM), jnp.float32),
    mesh=_sc_mesh,
    scratch_types=[
        pltpu.VMEM((_B_PER_W,), jnp.int32),
        pltpu.VMEM((_B_PER_W, Z_DIM), jnp.float32),
        pltpu.SemaphoreType.DMA,
    ],
)
def _sc_gather(y_hbm, idxpad_hbm, out_hbm, idx_v, rows_v, sem):
    wid = lax.axis_index("s") * 2 + lax.axis_index("c")
    pltpu.sync_copy(idxpad_hbm.at[pl.ds(wid * 8, _B_PER_W)], idx_v)
    pltpu.async_copy(y_hbm.at[idx_v], rows_v, sem).wait()
    pltpu.sync_copy(rows_v, out_hbm.at[pl.ds(wid * _B_PER_W, _B_PER_W)])


def kernel(indices, z, W, b):
    idx = indices.astype(jnp.int32)
    w_flat = W.reshape(ROWS, D_IN)
    b_flat = b.reshape(ROWS, 1)
    z_col = z.reshape(D_IN, 1)
    idx_pad = jnp.pad(idx.reshape(32, _B_PER_W),
                      ((0, 0), (0, 8 - _B_PER_W))).reshape(-1)
    y = _matvec(w_flat, z_col, b_flat).reshape(E, Z_DIM)
    return _sc_gather(y, idx_pad)
